# baseline (speedup 1.0000x reference)
import functools

import jax
import jax.numpy as jnp
from jax import lax
from jax.experimental import pallas as pl
from jax.experimental.pallas import tpu as pltpu
from jax.experimental.pallas import tpu_sc as plsc

NC = 2
NS = 16
LN = 16
NW = NC * NS
EB = 128
RB = 1000


def _sc_mesh():
    return plsc.VectorSubcoreMesh(
        core_axis_name="c", subcore_axis_name="s", num_cores=NC, num_subcores=NS
    )


def _row_chunks(N):
    ch = ((-(-N // NS)) + 7) // 8 * 8
    last = N - ch * (NS - 1)
    assert last > 0 and last % 8 == 0 and ch % 8 == 0
    return ch, last


def _tile_rows(s, N, fn):
    ch, last = _row_chunks(N)

    @pl.when(s < NS - 1)
    def _():
        fn(pl.multiple_of(s * ch, 8), ch)

    @pl.when(s == NS - 1)
    def _():
        fn((NS - 1) * ch, last)


def _spmm_partials(N, nb, D, ep, xw, zrows):
    nbg = (nb - 2) // 3
    assert nbg * 3 + 2 == nb
    gsems = [pltpu.SemaphoreType.DMA] * (3 if xw is not None else 0)

    @functools.partial(
        pl.kernel,
        out_type=(
            jax.ShapeDtypeStruct((N, D), jnp.float32),
            jax.ShapeDtypeStruct((N, D), jnp.float32),
        ),
        mesh=_sc_mesh(),
        compiler_params=pltpu.CompilerParams(
            needs_layout_passes=False, use_tc_tiling_on_sc=False),
        scratch_types=[
            pltpu.VMEM((3, EB), jnp.int32),
            pltpu.VMEM((3, EB), jnp.int32),
            pltpu.VMEM((3, EB), jnp.int32),
            pltpu.VMEM((EB, D), jnp.float32),
            pltpu.VMEM((EB, D), jnp.float32),
            pltpu.VMEM((EB, D), jnp.float32),
            pltpu.VMEM((EB,), jnp.float32),
            pltpu.VMEM_SHARED((N, D), jnp.float32),
            pltpu.SemaphoreType.DMA,
            pltpu.SemaphoreType.DMA,
            pltpu.SemaphoreType.DMA,
            pltpu.SemaphoreType.DMA,
            pltpu.SemaphoreType.DMA,
            pltpu.SemaphoreType.DMA,
        ] + gsems,
    )
    def k(*refs):
        if xw is not None:
            (ep_hbm, xw_hbm, z_hbm, outa, outb,
             ib0, ib1, ib2, r0, r1, r2, normv, acc,
             si0, si1, si2, ss0, ss1, ss2, sg0, sg1, sg2) = refs
            sgs = (sg0, sg1, sg2)
        else:
            (ep_hbm, z_hbm, outa, outb,
             ib0, ib1, ib2, r0, r1, r2, normv, acc,
             si0, si1, si2, ss0, ss1, ss2) = refs
            xw_hbm = None
        ibufs = (ib0, ib1, ib2)
        rows = (r0, r1, r2)
        sis = (si0, si1, si2)
        sss = (ss0, ss1, ss2)
        c = lax.axis_index("c")
        s = lax.axis_index("s")
        wid = c * NS + s
        _tile_rows(s, N, lambda off, sz: pltpu.sync_copy(
            z_hbm.at[pl.ds(0, sz), :], acc.at[pl.ds(off, sz), :]))
        plsc.subcore_barrier()

        def start_idx(i, b):
            pltpu.async_copy(ep_hbm.at[wid, i], ibufs[b], sis[b])

        def wait_idx(b):
            pltpu.make_async_copy(ep_hbm.at[wid, 0], ibufs[b], sis[b]).wait()

        def start_gather(b):
            pltpu.async_copy(xw_hbm.at[ibufs[b].at[0]], rows[b], sgs[b])

        def wait_gather(b):
            pltpu.make_async_copy(
                xw_hbm.at[ibufs[b].at[0]], rows[b], sgs[b]).wait()

        def start_scat(b):
            pltpu.async_copy(rows[b], acc.at[ibufs[b].at[1]], sss[b], add=True)

        def wait_scat(b):
            pltpu.make_async_copy(rows[b], acc.at[ibufs[b].at[1]], sss[b]).wait()

        def process(b):
            ib = ibufs[b]
            rb = rows[b]
            for g in range(EB // LN):
                gsl = pl.ds(g * LN, LN)
                normv[gsl] = plsc.bitcast(ib[2, gsl], jnp.float32)

            @plsc.parallel_loop(0, EB, step=16)
            def sbody(jj):
                for dj in range(16):
                    j = jj + dj
                    bj = plsc.load_gather(
                        normv, [jnp.zeros((LN,), jnp.int32) + j])
                    for f in range(D // LN):
                        fs = pl.ds(f * LN, LN)
                        if xw is None:
                            rb[j, fs] = bj
                        else:
                            rb[j, fs] = rb[j, fs] * bj

        def slot(i, b, nxt, prv, first):
            if xw is not None:
                wait_idx(nxt)
                start_gather(nxt)
                wait_gather(b)
            else:
                wait_idx(b)
            process(b)
            start_scat(b)
            if first:
                @pl.when(i >= 1)
                def _():
                    wait_scat(prv)
            else:
                wait_scat(prv)
            start_idx(i + 2, prv)

        start_idx(0, 0)
        start_idx(1, 1)
        if xw is not None:
            wait_idx(0)
            start_gather(0)

        def group(g, carry):
            i0 = g * 3
            slot(i0, 0, 1, 2, True)
            slot(i0 + 1, 1, 2, 0, False)
            slot(i0 + 2, 2, 0, 1, False)
            return carry

        lax.fori_loop(0, nbg, group, 0)

        if xw is not None:
            wait_idx(1)
            start_gather(1)
            wait_gather(0)
        else:
            wait_idx(0)
        process(0)
        start_scat(0)
        if xw is not None:
            wait_gather(1)
        else:
            wait_idx(1)
        process(1)
        start_scat(1)
        wait_scat(2)
        wait_scat(0)
        wait_scat(1)
        plsc.subcore_barrier()

        @pl.when(c == 0)
        def _():
            _tile_rows(s, N, lambda off, sz: pltpu.sync_copy(
                acc.at[pl.ds(off, sz), :], outa.at[pl.ds(off, sz), :]))

        @pl.when(c == 1)
        def _():
            _tile_rows(s, N, lambda off, sz: pltpu.sync_copy(
                acc.at[pl.ds(off, sz), :], outb.at[pl.ds(off, sz), :]))

    if xw is None:
        return k(ep, zrows)
    return k(ep, xw, zrows)


def _full(shape):
    return pl.BlockSpec(shape, lambda i: (0,) * len(shape))


def _rows(cols):
    return pl.BlockSpec((RB, cols), lambda i: (i, 0))


def _tc1(N, dega, degb, Y, W1e):
    def body(da, db, y, w, dis_o, xw_o):
        deg = da[:, :1] + db[:, :1] + 1.0
        dis = lax.rsqrt(deg)
        dis_o[...] = dis
        xw = jnp.dot(y[...], w[...], preferred_element_type=jnp.float32)
        xw_o[...] = xw * dis

    return pl.pallas_call(
        body,
        grid=(N // RB,),
        in_specs=[_rows(16), _rows(16), _rows(128), _full((128, 128))],
        out_specs=[_rows(1), _rows(128)],
        out_shape=[
            jax.ShapeDtypeStruct((N, 1), jnp.float32),
            jax.ShapeDtypeStruct((N, 128), jnp.float32),
        ],
    )(dega, degb, Y, W1e)


def _tc2(N, sa, sb, xw, dis, bias, W):
    din, dout = W.shape

    def body(a, b, x, d, bi, w, o):
        h = jnp.maximum((a[...] + b[...] + x[...]) * d[...] + bi[...], 0.0)
        o[...] = jnp.dot(h, w[...], preferred_element_type=jnp.float32) * d[...]

    return pl.pallas_call(
        body,
        grid=(N // RB,),
        in_specs=[_rows(din), _rows(din), _rows(din), _rows(1),
                  _full((1, din)), _full((din, dout))],
        out_specs=[_rows(dout)],
        out_shape=[jax.ShapeDtypeStruct((N, dout), jnp.float32)],
    )(sa, sb, xw, dis, bias.reshape(1, din), W)[0]


def _tc3(N, sa, sb, xw, dis, bml, eps, W1d):
    def body(a, b, x, d, bi, e, w, mu_o, lv_o, xw_o):
        g = (a[...] + b[...] + x[...]) * d[...] + bi[...]
        mu = g[:, :32]
        lv = g[:, 32:]
        z = mu + e[...] * jnp.exp(0.5 * lv)
        mu_o[...] = mu
        lv_o[...] = lv
        xw_o[...] = jnp.dot(
            z, w[...], preferred_element_type=jnp.float32) * d[...]

    return pl.pallas_call(
        body,
        grid=(N // RB,),
        in_specs=[_rows(64), _rows(64), _rows(64), _rows(1),
                  _full((1, 64)), _rows(32), _full((32, 128))],
        out_specs=[_rows(32), _rows(32), _rows(128)],
        out_shape=[
            jax.ShapeDtypeStruct((N, 32), jnp.float32),
            jax.ShapeDtypeStruct((N, 32), jnp.float32),
            jax.ShapeDtypeStruct((N, 128), jnp.float32),
        ],
    )(sa, sb, xw, dis, bml, eps, W1d)


def _tc5(N, n_ct, sa, sb, xw, dis, b2dp, xrefp):
    def body(a, b, x, d, bi, xr, b_o, y_o):
        logits = (a[...] + b[...] + x[...]) * d[...] + bi[...]
        col = lax.broadcasted_iota(jnp.int32, logits.shape, 1)
        mask = col < n_ct
        m = jnp.max(jnp.where(mask, logits, -jnp.inf), axis=1, keepdims=True)
        ex = jnp.where(mask, jnp.exp(logits - m), 0.0)
        bfull = ex / jnp.sum(ex, axis=1, keepdims=True)
        b_o[...] = bfull[:, :n_ct]
        y_o[...] = jnp.dot(bfull, xr[...], preferred_element_type=jnp.float32)

    return pl.pallas_call(
        body,
        grid=(N // RB,),
        in_specs=[_rows(32), _rows(32), _rows(32), _rows(1),
                  _full((1, 32)), _full((32, 128))],
        out_specs=[_rows(n_ct), _rows(128)],
        out_shape=[
            jax.ShapeDtypeStruct((N, n_ct), jnp.float32),
            jax.ShapeDtypeStruct((N, 128), jnp.float32),
        ],
    )(sa, sb, xw, dis, b2dp, xrefp)


def kernel(Y, edge_index, edge_weight, X_ref, W1e, b1e, Wmu, bmu, Wlv, blv,
           W1d, b1d, W2d, b2d):
    N, in_dim = Y.shape
    E = edge_index.shape[1]
    n_ct = W2d.shape[1]
    ch, _ = _row_chunks(N)

    src = edge_index[0]
    dst = edge_index[1]
    ew = edge_weight
    ept = E // NW
    nb = -(-ept // EB)
    while (nb - 2) % 3 != 0:
        nb += 1
    pad = nb * EB - ept

    spread = jnp.tile(jnp.arange(pad, dtype=jnp.int32)[None], (NW, 1)) % N

    def _tile_pack(x, padv):
        return jnp.concatenate(
            [x.reshape(NW, ept), padv], axis=1).reshape(NW, nb, EB)

    zpad = jnp.zeros((NW, pad), jnp.int32)
    ep = jnp.stack([
        _tile_pack(src, spread),
        _tile_pack(dst, spread),
        _tile_pack(lax.bitcast_convert_type(ew, jnp.int32), zpad),
    ], axis=2)

    z16 = jnp.zeros((ch, 16), jnp.float32)
    z128 = jnp.zeros((ch, 128), jnp.float32)
    z64 = jnp.zeros((ch, 64), jnp.float32)
    z32 = jnp.zeros((ch, 32), jnp.float32)

    Wml = jnp.concatenate([Wmu, Wlv], axis=1)
    bml = jnp.concatenate([bmu, blv]).reshape(1, 64)
    W2dp = jnp.pad(W2d, ((0, 0), (0, 32 - n_ct)))
    b2dp = jnp.pad(b2d, (0, 32 - n_ct)).reshape(1, 32)
    xrefp = jnp.pad(X_ref, ((0, 32 - n_ct), (0, 0)))
    eps = jax.random.normal(jax.random.key(42), (N, 32), jnp.float32)

    dega, degb = _spmm_partials(N, nb, 16, ep, None, z16)
    dis2d, xw1 = _tc1(N, dega, degb, Y, W1e)

    s1a, s1b = _spmm_partials(N, nb, 128, ep, xw1, z128)
    xw2 = _tc2(N, s1a, s1b, xw1, dis2d, b1e, Wml)

    s2a, s2b = _spmm_partials(N, nb, 64, ep, xw2, z64)
    mu, logvar, xw3 = _tc3(N, s2a, s2b, xw2, dis2d, bml, eps, W1d)

    s3a, s3b = _spmm_partials(N, nb, 128, ep, xw3, z128)
    xw4 = _tc2(N, s3a, s3b, xw3, dis2d, b1d, W2dp)

    s4a, s4b = _spmm_partials(N, nb, 32, ep, xw4, z32)
    Bout, Yhat = _tc5(N, n_ct, s4a, s4b, xw4, dis2d, b2dp, xrefp)

    return (Yhat, mu, logvar, Bout)


# speedup vs baseline: 1.0900x; 1.0900x over previous
import functools

import jax
import jax.numpy as jnp
from jax import lax
from jax.experimental import pallas as pl
from jax.experimental.pallas import tpu as pltpu
from jax.experimental.pallas import tpu_sc as plsc

NC = 2
NS = 16
LN = 16
NW = NC * NS
EB = 128
RB = 1000


def _sc_mesh():
    return plsc.VectorSubcoreMesh(
        core_axis_name="c", subcore_axis_name="s", num_cores=NC, num_subcores=NS
    )


def _row_chunks(N):
    ch = ((-(-N // NS)) + 7) // 8 * 8
    last = N - ch * (NS - 1)
    assert last > 0 and last % 8 == 0 and ch % 8 == 0
    return ch, last


def _tile_rows(s, N, fn):
    ch, last = _row_chunks(N)

    @pl.when(s < NS - 1)
    def _():
        fn(pl.multiple_of(s * ch, 8), ch)

    @pl.when(s == NS - 1)
    def _():
        fn((NS - 1) * ch, last)


def _spmm_partials(N, nb, D, ep, xw, zrows):
    nbg = (nb - 2) // 3
    assert nbg * 3 + 2 == nb
    gsems = [pltpu.SemaphoreType.DMA] * (3 if xw is not None else 0)

    @functools.partial(
        pl.kernel,
        out_type=(
            jax.ShapeDtypeStruct((N, D), jnp.float32),
            jax.ShapeDtypeStruct((N, D), jnp.float32),
        ),
        mesh=_sc_mesh(),
        compiler_params=pltpu.CompilerParams(
            needs_layout_passes=False, use_tc_tiling_on_sc=False),
        scratch_types=[
            pltpu.VMEM((3, EB), jnp.int32),
            pltpu.VMEM((3, EB), jnp.int32),
            pltpu.VMEM((3, EB), jnp.int32),
            pltpu.VMEM((EB, D), jnp.float32),
            pltpu.VMEM((EB, D), jnp.float32),
            pltpu.VMEM((EB, D), jnp.float32),
            pltpu.VMEM((EB,), jnp.float32),
            pltpu.VMEM_SHARED((N, D), jnp.float32),
            pltpu.SemaphoreType.DMA,
            pltpu.SemaphoreType.DMA,
            pltpu.SemaphoreType.DMA,
            pltpu.SemaphoreType.DMA,
            pltpu.SemaphoreType.DMA,
            pltpu.SemaphoreType.DMA,
        ] + gsems,
    )
    def k(*refs):
        if xw is not None:
            (ep_hbm, xw_hbm, z_hbm, outa, outb,
             ib0, ib1, ib2, r0, r1, r2, normv, acc,
             si0, si1, si2, ss0, ss1, ss2, sg0, sg1, sg2) = refs
            sgs = (sg0, sg1, sg2)
        else:
            (ep_hbm, z_hbm, outa, outb,
             ib0, ib1, ib2, r0, r1, r2, normv, acc,
             si0, si1, si2, ss0, ss1, ss2) = refs
            xw_hbm = None
        ibufs = (ib0, ib1, ib2)
        rows = (r0, r1, r2)
        sis = (si0, si1, si2)
        sss = (ss0, ss1, ss2)
        c = lax.axis_index("c")
        s = lax.axis_index("s")
        wid = c * NS + s
        _tile_rows(s, N, lambda off, sz: pltpu.sync_copy(
            z_hbm.at[pl.ds(0, sz), :], acc.at[pl.ds(off, sz), :]))
        plsc.subcore_barrier()

        def start_idx(i, b):
            pltpu.async_copy(ep_hbm.at[wid, i], ibufs[b], sis[b])

        def wait_idx(b):
            pltpu.make_async_copy(ep_hbm.at[wid, 0], ibufs[b], sis[b]).wait()

        def start_gather(b):
            pltpu.async_copy(xw_hbm.at[ibufs[b].at[0]], rows[b], sgs[b])

        def wait_gather(b):
            pltpu.make_async_copy(
                xw_hbm.at[ibufs[b].at[0]], rows[b], sgs[b]).wait()

        def start_scat(b):
            pltpu.async_copy(rows[b], acc.at[ibufs[b].at[1]], sss[b], add=True)

        def wait_scat(b):
            pltpu.make_async_copy(rows[b], acc.at[ibufs[b].at[1]], sss[b]).wait()

        def process(b):
            ib = ibufs[b]
            rb = rows[b]
            for g in range(EB // LN):
                gsl = pl.ds(g * LN, LN)
                normv[gsl] = plsc.bitcast(ib[2, gsl], jnp.float32)

            @plsc.parallel_loop(0, EB, step=4)
            def sbody(jj):
                for dj in range(4):
                    j = jj + dj
                    bj = plsc.load_gather(
                        normv, [jnp.zeros((LN,), jnp.int32) + j])
                    for f in range(D // LN):
                        fs = pl.ds(f * LN, LN)
                        if xw is None:
                            rb[j, fs] = bj
                        else:
                            rb[j, fs] = rb[j, fs] * bj

        def slot(i, b, nxt, prv, first):
            if xw is not None:
                wait_idx(nxt)
                start_gather(nxt)
                wait_gather(b)
            else:
                wait_idx(b)
            process(b)
            start_scat(b)
            if first:
                @pl.when(i >= 1)
                def _():
                    wait_scat(prv)
            else:
                wait_scat(prv)
            start_idx(i + 2, prv)

        start_idx(0, 0)
        start_idx(1, 1)
        if xw is not None:
            wait_idx(0)
            start_gather(0)

        def group(g, carry):
            i0 = g * 3
            slot(i0, 0, 1, 2, True)
            slot(i0 + 1, 1, 2, 0, False)
            slot(i0 + 2, 2, 0, 1, False)
            return carry

        lax.fori_loop(0, nbg, group, 0)

        if xw is not None:
            wait_idx(1)
            start_gather(1)
            wait_gather(0)
        else:
            wait_idx(0)
        process(0)
        start_scat(0)
        if xw is not None:
            wait_gather(1)
        else:
            wait_idx(1)
        process(1)
        start_scat(1)
        wait_scat(2)
        wait_scat(0)
        wait_scat(1)
        plsc.subcore_barrier()

        @pl.when(c == 0)
        def _():
            _tile_rows(s, N, lambda off, sz: pltpu.sync_copy(
                acc.at[pl.ds(off, sz), :], outa.at[pl.ds(off, sz), :]))

        @pl.when(c == 1)
        def _():
            _tile_rows(s, N, lambda off, sz: pltpu.sync_copy(
                acc.at[pl.ds(off, sz), :], outb.at[pl.ds(off, sz), :]))

    if xw is None:
        return k(ep, zrows)
    return k(ep, xw, zrows)


def _full(shape):
    return pl.BlockSpec(shape, lambda i: (0,) * len(shape))


def _rows(cols):
    return pl.BlockSpec((RB, cols), lambda i: (i, 0))


def _tc1(N, dega, degb, Y, W1e):
    def body(da, db, y, w, dis_o, xw_o):
        deg = da[:, :1] + db[:, :1] + 1.0
        dis = lax.rsqrt(deg)
        dis_o[...] = dis
        xw = jnp.dot(y[...], w[...], preferred_element_type=jnp.float32)
        xw_o[...] = xw * dis

    return pl.pallas_call(
        body,
        grid=(N // RB,),
        in_specs=[_rows(16), _rows(16), _rows(128), _full((128, 128))],
        out_specs=[_rows(1), _rows(128)],
        out_shape=[
            jax.ShapeDtypeStruct((N, 1), jnp.float32),
            jax.ShapeDtypeStruct((N, 128), jnp.float32),
        ],
    )(dega, degb, Y, W1e)


def _tc2(N, sa, sb, xw, dis, bias, W):
    din, dout = W.shape

    def body(a, b, x, d, bi, w, o):
        h = jnp.maximum((a[...] + b[...] + x[...]) * d[...] + bi[...], 0.0)
        o[...] = jnp.dot(h, w[...], preferred_element_type=jnp.float32) * d[...]

    return pl.pallas_call(
        body,
        grid=(N // RB,),
        in_specs=[_rows(din), _rows(din), _rows(din), _rows(1),
                  _full((1, din)), _full((din, dout))],
        out_specs=[_rows(dout)],
        out_shape=[jax.ShapeDtypeStruct((N, dout), jnp.float32)],
    )(sa, sb, xw, dis, bias.reshape(1, din), W)[0]


def _tc3(N, sa, sb, xw, dis, bml, eps, W1d):
    def body(a, b, x, d, bi, e, w, mu_o, lv_o, xw_o):
        g = (a[...] + b[...] + x[...]) * d[...] + bi[...]
        mu = g[:, :32]
        lv = g[:, 32:]
        z = mu + e[...] * jnp.exp(0.5 * lv)
        mu_o[...] = mu
        lv_o[...] = lv
        xw_o[...] = jnp.dot(
            z, w[...], preferred_element_type=jnp.float32) * d[...]

    return pl.pallas_call(
        body,
        grid=(N // RB,),
        in_specs=[_rows(64), _rows(64), _rows(64), _rows(1),
                  _full((1, 64)), _rows(32), _full((32, 128))],
        out_specs=[_rows(32), _rows(32), _rows(128)],
        out_shape=[
            jax.ShapeDtypeStruct((N, 32), jnp.float32),
            jax.ShapeDtypeStruct((N, 32), jnp.float32),
            jax.ShapeDtypeStruct((N, 128), jnp.float32),
        ],
    )(sa, sb, xw, dis, bml, eps, W1d)


def _tc5(N, n_ct, sa, sb, xw, dis, b2dp, xrefp):
    def body(a, b, x, d, bi, xr, b_o, y_o):
        logits = (a[...] + b[...] + x[...]) * d[...] + bi[...]
        col = lax.broadcasted_iota(jnp.int32, logits.shape, 1)
        mask = col < n_ct
        m = jnp.max(jnp.where(mask, logits, -jnp.inf), axis=1, keepdims=True)
        ex = jnp.where(mask, jnp.exp(logits - m), 0.0)
        bfull = ex / jnp.sum(ex, axis=1, keepdims=True)
        b_o[...] = bfull[:, :n_ct]
        y_o[...] = jnp.dot(bfull, xr[...], preferred_element_type=jnp.float32)

    return pl.pallas_call(
        body,
        grid=(N // RB,),
        in_specs=[_rows(32), _rows(32), _rows(32), _rows(1),
                  _full((1, 32)), _full((32, 128))],
        out_specs=[_rows(n_ct), _rows(128)],
        out_shape=[
            jax.ShapeDtypeStruct((N, n_ct), jnp.float32),
            jax.ShapeDtypeStruct((N, 128), jnp.float32),
        ],
    )(sa, sb, xw, dis, b2dp, xrefp)


def kernel(Y, edge_index, edge_weight, X_ref, W1e, b1e, Wmu, bmu, Wlv, blv,
           W1d, b1d, W2d, b2d):
    N, in_dim = Y.shape
    E = edge_index.shape[1]
    n_ct = W2d.shape[1]
    ch, _ = _row_chunks(N)

    src = edge_index[0]
    dst = edge_index[1]
    ew = edge_weight
    ept = E // NW
    nb = -(-ept // EB)
    while (nb - 2) % 3 != 0:
        nb += 1
    pad = nb * EB - ept

    spread = jnp.tile(jnp.arange(pad, dtype=jnp.int32)[None], (NW, 1)) % N

    def _tile_pack(x, padv):
        return jnp.concatenate(
            [x.reshape(NW, ept), padv], axis=1).reshape(NW, nb, EB)

    zpad = jnp.zeros((NW, pad), jnp.int32)
    ep = jnp.stack([
        _tile_pack(src, spread),
        _tile_pack(dst, spread),
        _tile_pack(lax.bitcast_convert_type(ew, jnp.int32), zpad),
    ], axis=2)

    z16 = jnp.zeros((ch, 16), jnp.float32)
    z128 = jnp.zeros((ch, 128), jnp.float32)
    z64 = jnp.zeros((ch, 64), jnp.float32)
    z32 = jnp.zeros((ch, 32), jnp.float32)

    Wml = jnp.concatenate([Wmu, Wlv], axis=1)
    bml = jnp.concatenate([bmu, blv]).reshape(1, 64)
    W2dp = jnp.pad(W2d, ((0, 0), (0, 32 - n_ct)))
    b2dp = jnp.pad(b2d, (0, 32 - n_ct)).reshape(1, 32)
    xrefp = jnp.pad(X_ref, ((0, 32 - n_ct), (0, 0)))
    eps = jax.random.normal(jax.random.key(42), (N, 32), jnp.float32)

    dega, degb = _spmm_partials(N, nb, 16, ep, None, z16)
    dis2d, xw1 = _tc1(N, dega, degb, Y, W1e)

    s1a, s1b = _spmm_partials(N, nb, 128, ep, xw1, z128)
    xw2 = _tc2(N, s1a, s1b, xw1, dis2d, b1e, Wml)

    s2a, s2b = _spmm_partials(N, nb, 64, ep, xw2, z64)
    mu, logvar, xw3 = _tc3(N, s2a, s2b, xw2, dis2d, bml, eps, W1d)

    s3a, s3b = _spmm_partials(N, nb, 128, ep, xw3, z128)
    xw4 = _tc2(N, s3a, s3b, xw3, dis2d, b1d, W2dp)

    s4a, s4b = _spmm_partials(N, nb, 32, ep, xw4, z32)
    Bout, Yhat = _tc5(N, n_ct, s4a, s4b, xw4, dis2d, b2dp, xrefp)

    return (Yhat, mu, logvar, Bout)
